# Initial kernel scaffold; baseline (speedup 1.0000x reference)
import jax
import jax.numpy as jnp
from jax.experimental import pallas as pl


def kernel(hv, edge_index, he, W_msg, b_msg, W_ih, W_hh, b_ih, b_hh):
    raise NotImplementedError("write your pallas kernel here")


# speedup vs baseline: 4.7059x; 4.7059x over previous
import functools

import jax
import jax.numpy as jnp
from jax import lax
from jax.experimental import pallas as pl
from jax.experimental.pallas import tpu as pltpu
from jax.experimental.pallas import tpu_sc as plsc

NC = 2
NS = 16
NW = NC * NS
L = 16
CH = 128


def _sc_scatter_rows(N_PAD, D, EPT, with_aux):
    RPT = N_PAD // NS
    NCHUNK = EPT // CH
    mesh = plsc.VectorSubcoreMesh(
        core_axis_name="c", subcore_axis_name="s",
        num_cores=NC, num_subcores=NS)

    out_type = [jax.ShapeDtypeStruct((NC, N_PAD, D), jnp.float32)]
    if with_aux:
        out_type += [jax.ShapeDtypeStruct((NW, N_PAD), jnp.float32)] * 2

    scratch = [
        pltpu.VMEM_SHARED((N_PAD, D), jnp.float32),
        pltpu.VMEM((CH,), jnp.int32),
        pltpu.VMEM((CH,), jnp.int32),
        pltpu.VMEM((CH, D), jnp.float32),
        pltpu.SemaphoreType.DMA,
    ]
    if with_aux:
        scratch += [
            pltpu.VMEM((CH,), jnp.float32),
            pltpu.VMEM((N_PAD,), jnp.float32),
            pltpu.VMEM((N_PAD,), jnp.float32),
        ]

    @functools.partial(pl.kernel, mesh=mesh, out_type=out_type,
                       scratch_types=scratch,
                       compiler_params=pltpu.CompilerParams(
                           needs_layout_passes=False))
    def body(hv_hbm, src_hbm, dst_hbm, he_hbm, zrows_hbm, zvec_hbm, *rest):
        if with_aux:
            (s_out, deg_out, she_out,
             acc, src_v, dst_v, rows, sem, he_v, deg_v, she_v) = rest
        else:
            s_out, acc, src_v, dst_v, rows, sem = rest
        c = lax.axis_index("c")
        s = lax.axis_index("s")
        wid = s * NC + c

        pltpu.sync_copy(zrows_hbm, acc.at[pl.ds(s * RPT, RPT)])
        if with_aux:
            pltpu.sync_copy(zvec_hbm, deg_v)
            pltpu.sync_copy(zvec_hbm, she_v)
        plsc.subcore_barrier()

        ebase = wid * EPT
        ones = jnp.ones((L,), jnp.float32)

        def chunk(j, carry):
            off = ebase + j * CH
            pltpu.sync_copy(src_hbm.at[pl.ds(off, CH)], src_v)
            pltpu.sync_copy(dst_hbm.at[pl.ds(off, CH)], dst_v)
            pltpu.async_copy(hv_hbm.at[src_v], rows, sem).wait()
            pltpu.sync_copy(rows, acc.at[dst_v], add=True)
            if with_aux:
                pltpu.sync_copy(he_hbm.at[pl.ds(off, CH)], he_v)
                for k in range(CH // L):
                    dk = dst_v[pl.ds(k * L, L)]
                    plsc.addupdate_scatter(deg_v, [dk], ones)
                    plsc.addupdate_scatter(she_v, [dk], he_v[pl.ds(k * L, L)])
            return carry

        lax.fori_loop(0, NCHUNK, chunk, 0)
        plsc.subcore_barrier()

        pltpu.sync_copy(acc.at[pl.ds(s * RPT, RPT)],
                        s_out.at[c, pl.ds(s * RPT, RPT)])
        if with_aux:
            pltpu.sync_copy(deg_v, deg_out.at[wid])
            pltpu.sync_copy(she_v, she_out.at[wid])

    return body


def _tc_update(N, D, BN):
    D2, D3 = 2 * D, 3 * D
    grid = (pl.cdiv(N, BN),)

    def body(hv_ref, sp_ref, degp_ref, shep_ref, wdt_ref, wst_ref, we_ref,
             bm_ref, wiht_ref, whht_ref, bih_ref, bhh_ref, out_ref):
        hvb = hv_ref[...]
        hv16 = hvb.astype(jnp.bfloat16)
        sp = sp_ref[...]
        Sb = sp[0] + sp[1]
        onesw = jnp.ones((NW, 1), jnp.float32)
        dn = (((0,), (0,)), ((), ()))
        hi = lax.Precision.HIGHEST
        degc = lax.dot_general(degp_ref[...], onesw, dn, precision=hi,
                               preferred_element_type=jnp.float32)
        shec = lax.dot_general(shep_ref[...], onesw, dn, precision=hi,
                               preferred_element_type=jnp.float32)
        a = jnp.dot(hv16, wdt_ref[...],
                    preferred_element_type=jnp.float32) * degc
        a = a + jnp.dot(Sb, wst_ref[...], precision=hi,
                        preferred_element_type=jnp.float32)
        a = a + shec * we_ref[...] + degc * bm_ref[...]
        gi = jnp.dot(a.astype(jnp.bfloat16), wiht_ref[...],
                     preferred_element_type=jnp.float32) + bih_ref[...]
        gh = jnp.dot(hv16, whht_ref[...],
                     preferred_element_type=jnp.float32) + bhh_ref[...]
        r = jax.nn.sigmoid(gi[:, :D] + gh[:, :D])
        z = jax.nn.sigmoid(gi[:, D:D2] + gh[:, D:D2])
        n = jnp.tanh(gi[:, D2:] + r * gh[:, D2:])
        out_ref[...] = (1.0 - z) * n + z * hvb

    c0 = lambda i: (0, 0)
    return pl.pallas_call(
        body,
        grid=grid,
        in_specs=[
            pl.BlockSpec((BN, D), lambda i: (i, 0)),
            pl.BlockSpec((NC, BN, D), lambda i: (0, i, 0)),
            pl.BlockSpec((NW, BN), lambda i: (0, i)),
            pl.BlockSpec((NW, BN), lambda i: (0, i)),
            pl.BlockSpec((D, D2), c0),
            pl.BlockSpec((D, D2), c0),
            pl.BlockSpec((1, D2), c0),
            pl.BlockSpec((1, D2), c0),
            pl.BlockSpec((D2, D3), c0),
            pl.BlockSpec((D, D3), c0),
            pl.BlockSpec((1, D3), c0),
            pl.BlockSpec((1, D3), c0),
        ],
        out_specs=pl.BlockSpec((BN, D), lambda i: (i, 0)),
        out_shape=jax.ShapeDtypeStruct((N, D), jnp.float32),
    )


def kernel(hv, edge_index, he, W_msg, b_msg, W_ih, W_hh, b_ih, b_hh):
    N, D = hv.shape
    E = edge_index.shape[1]
    T = W_msg.shape[0]

    N_PAD = ((N + NW * 8 - 1) // (NW * 8)) * (NW * 8)
    EPT = ((E + NW - 1) // NW + CH - 1) // CH * CH
    E_PAD = EPT * NW

    def b16(x):
        return lax.reduce_precision(x, exponent_bits=8, mantissa_bits=7)

    order = jnp.argsort(edge_index[1], stable=True)
    src = edge_index[0][order]
    dst = edge_index[1][order]
    he_s = b16(he[:, 0])[order]
    pad = E_PAD - E
    src_p = jnp.concatenate([src, jnp.zeros((pad,), jnp.int32)])
    dst_p = jnp.concatenate([dst, jnp.full((pad,), N_PAD - 8, jnp.int32)])
    he_p = jnp.concatenate([he_s, jnp.zeros((pad,), jnp.float32)])
    zrows = jnp.zeros((N_PAD // NS, D), jnp.float32)
    zvec = jnp.zeros((N_PAD,), jnp.float32)

    sc_aux = _sc_scatter_rows(N_PAD, D, EPT, with_aux=True)
    sc_plain = _sc_scatter_rows(N_PAD, D, EPT, with_aux=False)
    tc = _tc_update(N, D, 2048)

    degp = shep = None
    for t in range(T):
        hv16f = b16(hv)
        if t == 0:
            S, degp, shep = sc_aux(hv16f, src_p, dst_p, he_p, zrows, zvec)
        else:
            (S,) = sc_plain(hv16f, src_p, dst_p, he_p, zrows, zvec)
        wdt = W_msg[t, :, :D].T.astype(jnp.bfloat16)
        wst = b16(W_msg[t, :, D:2 * D].T)
        we = b16(W_msg[t, :, 2 * D][None, :])
        bm = b_msg[t][None, :]
        wiht = W_ih[t].T.astype(jnp.bfloat16)
        whht = W_hh[t].T.astype(jnp.bfloat16)
        bih = b_ih[t][None, :]
        bhh = b_hh[t][None, :]
        hv = tc(hv, S, degp, shep,
                wdt, wst, we, bm, wiht, whht, bih, bhh)
    return hv
